# Initial kernel scaffold; baseline (speedup 1.0000x reference)
import jax
import jax.numpy as jnp
from jax.experimental import pallas as pl


def kernel(x, edge_index, W, b, eps):
    raise NotImplementedError("write your pallas kernel here")


# speedup vs baseline: 5.1870x; 5.1870x over previous
import functools

import jax
import jax.numpy as jnp
from jax import lax
from jax.experimental import pallas as pl
from jax.experimental.pallas import tpu as pltpu
from jax.experimental.pallas import tpu_sc as plsc

B, C, N, K = 1, 128, 10000, 32
C_OUT = 128

NW = 32
BPW = 320
NPAD = NW * BPW
CH = 64
CPW = BPW // CH
NB = 1024


def _sc_body(xT_hbm, idx_hbm, out_hbm, idx_v, buf_v, acc_v, sem0, sem1):
    wid = lax.axis_index("s") * 2 + lax.axis_index("c")
    base = wid * BPW
    pltpu.sync_copy(idx_hbm.at[wid], idx_v)
    sems = (sem0, sem1)

    def chunk_body(c, carry):
        base_c = base + c * CH
        pltpu.sync_copy(xT_hbm.at[pl.ds(base_c, CH)], acc_v)
        handles = [None, None]
        handles[0] = pltpu.async_copy(
            xT_hbm.at[idx_v.at[0, c]], buf_v.at[0], sem0)
        for k in range(K):
            slot = k % 2
            handles[slot].wait()
            if k + 1 < K:
                nslot = (k + 1) % 2
                handles[nslot] = pltpu.async_copy(
                    xT_hbm.at[idx_v.at[k + 1, c]], buf_v.at[nslot], sems[nslot])

            def acc_body(n, carry2):
                for j in range(C // 16):
                    plsc.addupdate(
                        acc_v.at[n, pl.ds(j * 16, 16)],
                        buf_v[slot, n, pl.ds(j * 16, 16)])
                return carry2

            lax.fori_loop(0, CH, acc_body, 0, unroll=2)
        pltpu.sync_copy(acc_v, out_hbm.at[pl.ds(base_c, CH)])
        return carry

    lax.fori_loop(0, CPW, chunk_body, 0)


@functools.partial(
    pl.kernel,
    mesh=plsc.VectorSubcoreMesh(core_axis_name="c", subcore_axis_name="s"),
    out_type=jax.ShapeDtypeStruct((NPAD, C), jnp.float32),
    scratch_types=[
        pltpu.VMEM((K, CPW, CH), jnp.int32),
        pltpu.VMEM((2, CH, C), jnp.float32),
        pltpu.VMEM((CH, C), jnp.float32),
        pltpu.SemaphoreType.DMA,
        pltpu.SemaphoreType.DMA,
    ],
)
def _sc_gather_sum(xT_hbm, idx_hbm, out_hbm, idx_v, buf_v, acc_v, sem0, sem1):
    _sc_body(xT_hbm, idx_hbm, out_hbm, idx_v, buf_v, acc_v, sem0, sem1)


def _tc_body(eps_ref, w_ref, x_ref, s_ref, b_ref, o_ref):
    ws = lax.dot_general(w_ref[...], s_ref[...], (((1,), (1,)), ((), ())),
                         preferred_element_type=jnp.float32)
    wx = jnp.dot(w_ref[...], x_ref[...], preferred_element_type=jnp.float32)
    o_ref[...] = jnp.maximum(ws + eps_ref[0, 0] * wx + b_ref[...], 0.0)


_tc_mm = pl.pallas_call(
    _tc_body,
    grid=(NPAD // NB,),
    in_specs=[
        pl.BlockSpec((1, 1), lambda i: (0, 0)),
        pl.BlockSpec((C_OUT, C), lambda i: (0, 0)),
        pl.BlockSpec((C, NB), lambda i: (0, i)),
        pl.BlockSpec((NB, C), lambda i: (i, 0)),
        pl.BlockSpec((C_OUT, 1), lambda i: (0, 0)),
    ],
    out_specs=pl.BlockSpec((C_OUT, NB), lambda i: (0, i)),
    out_shape=jax.ShapeDtypeStruct((C_OUT, NPAD), jnp.float32),
)


def kernel(x, edge_index, W, b, eps):
    xm = x.reshape(C, N)
    xp = jnp.pad(xm, ((0, 0), (0, NPAD - N)))
    xT = jnp.pad(xm.T, ((0, NPAD - N), (0, 0)))
    idx = edge_index[0, 0]
    idxp = jnp.pad(idx, ((0, NPAD - N), (0, 0)))
    idx4 = jnp.transpose(idxp.T.reshape(K, NW, CPW, CH), (1, 0, 2, 3))
    s = _sc_gather_sum(xT, idx4)
    out = _tc_mm(eps.reshape(1, 1), W, xp, s, b.reshape(C_OUT, 1))
    return out[:, :N].reshape(1, C_OUT, N, 1)


# speedup vs baseline: 5.6718x; 1.0935x over previous
import functools

import jax
import jax.numpy as jnp
from jax import lax
from jax.experimental import pallas as pl
from jax.experimental.pallas import tpu as pltpu
from jax.experimental.pallas import tpu_sc as plsc

B, C, N, K = 1, 128, 10000, 32
C_OUT = 128

NW = 32
BPW = 320
NPAD = NW * BPW
CH = 64
CPW = BPW // CH
NB = 1024


def _sc_body(xT_hbm, idx_hbm, out_hbm, idx_v, buf_v, acc_v, sem0, sem1):
    wid = lax.axis_index("s") * 2 + lax.axis_index("c")
    base = wid * BPW
    pltpu.sync_copy(idx_hbm.at[wid], idx_v)
    sems = (sem0, sem1)

    def chunk_body(c, carry):
        base_c = base + c * CH
        pltpu.sync_copy(xT_hbm.at[pl.ds(base_c, CH)], acc_v)
        handles = [
            pltpu.async_copy(xT_hbm.at[idx_v.at[k, c]], acc_v, sem0, add=True)
            for k in range(K)
        ]
        for h in handles:
            h.wait()
        pltpu.sync_copy(acc_v, out_hbm.at[pl.ds(base_c, CH)])
        return carry

    lax.fori_loop(0, CPW, chunk_body, 0)


@functools.partial(
    pl.kernel,
    mesh=plsc.VectorSubcoreMesh(core_axis_name="c", subcore_axis_name="s"),
    out_type=jax.ShapeDtypeStruct((NPAD, C), jnp.float32),
    scratch_types=[
        pltpu.VMEM((K, CPW, CH), jnp.int32),
        pltpu.VMEM((2, CH, C), jnp.float32),
        pltpu.VMEM((CH, C), jnp.float32),
        pltpu.SemaphoreType.DMA,
        pltpu.SemaphoreType.DMA,
    ],
)
def _sc_gather_sum(xT_hbm, idx_hbm, out_hbm, idx_v, buf_v, acc_v, sem0, sem1):
    _sc_body(xT_hbm, idx_hbm, out_hbm, idx_v, buf_v, acc_v, sem0, sem1)


def _tc_body(eps_ref, w_ref, x_ref, s_ref, b_ref, o_ref):
    ws = lax.dot_general(w_ref[...], s_ref[...], (((1,), (1,)), ((), ())),
                         preferred_element_type=jnp.float32)
    wx = jnp.dot(w_ref[...], x_ref[...], preferred_element_type=jnp.float32)
    o_ref[...] = jnp.maximum(ws + eps_ref[0, 0] * wx + b_ref[...], 0.0)


_tc_mm = pl.pallas_call(
    _tc_body,
    grid=(NPAD // NB,),
    in_specs=[
        pl.BlockSpec((1, 1), lambda i: (0, 0)),
        pl.BlockSpec((C_OUT, C), lambda i: (0, 0)),
        pl.BlockSpec((C, NB), lambda i: (0, i)),
        pl.BlockSpec((NB, C), lambda i: (i, 0)),
        pl.BlockSpec((C_OUT, 1), lambda i: (0, 0)),
    ],
    out_specs=pl.BlockSpec((C_OUT, NB), lambda i: (0, i)),
    out_shape=jax.ShapeDtypeStruct((C_OUT, NPAD), jnp.float32),
)


def kernel(x, edge_index, W, b, eps):
    xm = x.reshape(C, N)
    xp = jnp.pad(xm, ((0, 0), (0, NPAD - N)))
    xT = jnp.pad(xm.T, ((0, NPAD - N), (0, 0)))
    idx = edge_index[0, 0]
    idxp = jnp.pad(idx, ((0, NPAD - N), (0, 0)))
    idx4 = jnp.transpose(idxp.T.reshape(K, NW, CPW, CH), (1, 0, 2, 3))
    s = _sc_gather_sum(xT, idx4)
    out = _tc_mm(eps.reshape(1, 1), W, xp, s, b.reshape(C_OUT, 1))
    return out[:, :N].reshape(1, C_OUT, N, 1)


# speedup vs baseline: 25.6239x; 4.5178x over previous
import functools

import jax
import jax.numpy as jnp
from jax import lax
from jax.experimental import pallas as pl
from jax.experimental.pallas import tpu as pltpu
from jax.experimental.pallas import tpu_sc as plsc

B, C, N, K = 1, 128, 10000, 32
C_OUT = 128

NW = 32
BPW = 320
NPAD = NW * BPW
CH = 64
CPW = BPW // CH
NB = 1024


def _sc_body(xT_hbm, idx_hbm, out_hbm, idx_v, spx, acc_v, sem0, sem1):
    sid = lax.axis_index("s")
    wid = sid * 2 + lax.axis_index("c")
    base = wid * BPW
    pltpu.sync_copy(idx_hbm.at[wid], idx_v)
    rps = NPAD // 16
    pltpu.sync_copy(xT_hbm.at[pl.ds(sid * rps, rps)],
                    spx.at[pl.ds(sid * rps, rps)])
    plsc.subcore_barrier()

    def chunk_body(c, carry):
        base_c = base + c * CH
        pltpu.sync_copy(spx.at[pl.ds(base_c, CH)], acc_v)
        handles = [
            pltpu.async_copy(spx.at[idx_v.at[k, c]], acc_v, sem0, add=True)
            for k in range(K)
        ]
        for h in handles:
            h.wait()
        pltpu.sync_copy(acc_v, out_hbm.at[pl.ds(base_c, CH)])
        return carry

    lax.fori_loop(0, CPW, chunk_body, 0)


@functools.partial(
    pl.kernel,
    mesh=plsc.VectorSubcoreMesh(core_axis_name="c", subcore_axis_name="s"),
    out_type=jax.ShapeDtypeStruct((NPAD, C), jnp.float32),
    scratch_types=[
        pltpu.VMEM((K, CPW, CH), jnp.int32),
        pltpu.VMEM_SHARED((NPAD, C), jnp.float32),
        pltpu.VMEM((CH, C), jnp.float32),
        pltpu.SemaphoreType.DMA,
        pltpu.SemaphoreType.DMA,
    ],
)
def _sc_gather_sum(xT_hbm, idx_hbm, out_hbm, idx_v, spx, acc_v, sem0, sem1):
    _sc_body(xT_hbm, idx_hbm, out_hbm, idx_v, spx, acc_v, sem0, sem1)


def _tc_body(eps_ref, w_ref, x_ref, s_ref, b_ref, o_ref):
    ws = lax.dot_general(w_ref[...], s_ref[...], (((1,), (1,)), ((), ())),
                         preferred_element_type=jnp.float32)
    wx = jnp.dot(w_ref[...], x_ref[...], preferred_element_type=jnp.float32)
    o_ref[...] = jnp.maximum(ws + eps_ref[0, 0] * wx + b_ref[...], 0.0)


_tc_mm = pl.pallas_call(
    _tc_body,
    grid=(NPAD // NB,),
    in_specs=[
        pl.BlockSpec((1, 1), lambda i: (0, 0)),
        pl.BlockSpec((C_OUT, C), lambda i: (0, 0)),
        pl.BlockSpec((C, NB), lambda i: (0, i)),
        pl.BlockSpec((NB, C), lambda i: (i, 0)),
        pl.BlockSpec((C_OUT, 1), lambda i: (0, 0)),
    ],
    out_specs=pl.BlockSpec((C_OUT, NB), lambda i: (0, i)),
    out_shape=jax.ShapeDtypeStruct((C_OUT, NPAD), jnp.float32),
)


def kernel(x, edge_index, W, b, eps):
    xm = x.reshape(C, N)
    xp = jnp.pad(xm, ((0, 0), (0, NPAD - N)))
    xT = jnp.pad(xm.T, ((0, NPAD - N), (0, 0)))
    idx = edge_index[0, 0]
    idxp = jnp.pad(idx, ((0, NPAD - N), (0, 0)))
    idx4 = jnp.transpose(idxp.T.reshape(K, NW, CPW, CH), (1, 0, 2, 3))
    s = _sc_gather_sum(xT, idx4)
    out = _tc_mm(eps.reshape(1, 1), W, xp, s, b.reshape(C_OUT, 1))
    return out[:, :N].reshape(1, C_OUT, N, 1)
